# Initial kernel scaffold; baseline (speedup 1.0000x reference)
import jax
import jax.numpy as jnp
from jax.experimental import pallas as pl


def kernel(lambda_mat, adj_mat, norm_adj, user_sv, item_sv):
    raise NotImplementedError("write your pallas kernel here")


# speedup vs baseline: 2.7375x; 2.7375x over previous
import functools

import jax
import jax.numpy as jnp
from jax.experimental import pallas as pl

N_USERS = 16384
N_ITEMS = 2048
RANK = 128
BU = 1024
MM_DTYPE = jnp.bfloat16


def _b_pass_kernel(usv_ref, adj_ref, b_ref):
    i = pl.program_id(0)
    usv = usv_ref[...].astype(MM_DTYPE)
    adj = adj_ref[...].astype(MM_DTYPE)
    part = jax.lax.dot_general(
        usv, adj, (((0,), (0,)), ((), ())),
        preferred_element_type=jnp.float32)

    @pl.when(i == 0)
    def _init():
        b_ref[...] = part

    @pl.when(i > 0)
    def _acc():
        b_ref[...] += part


def _rating_pass_kernel(lam_ref, isv_ref, nadj_ref, b_ref, out_ref):
    isv_s = (isv_ref[...] * (1.0 / lam_ref[...])).astype(MM_DTYPE)
    nadj = nadj_ref[...].astype(MM_DTYPE)
    c = jax.lax.dot_general(
        nadj, isv_s, (((1,), (0,)), ((), ())),
        preferred_element_type=jnp.float32).astype(MM_DTYPE)
    b = b_ref[...].astype(MM_DTYPE)
    r = jax.lax.dot_general(
        c, b, (((1,), (0,)), ((), ())),
        preferred_element_type=jnp.float32)
    out_ref[...] = r.astype(jnp.bfloat16)


@functools.partial(jax.jit, static_argnames=("interpret",))
def kernel(lambda_mat, adj_mat, norm_adj, user_sv, item_sv, interpret=False):
    n_blocks = N_USERS // BU

    b_mat = pl.pallas_call(
        _b_pass_kernel,
        grid=(n_blocks,),
        in_specs=[
            pl.BlockSpec((BU, RANK), lambda i: (i, 0)),
            pl.BlockSpec((BU, N_ITEMS), lambda i: (i, 0)),
        ],
        out_specs=pl.BlockSpec((RANK, N_ITEMS), lambda i: (0, 0)),
        out_shape=jax.ShapeDtypeStruct((RANK, N_ITEMS), jnp.float32),
        interpret=interpret,
    )(user_sv, adj_mat)

    lam_row = lambda_mat.reshape(1, RANK)
    rating = pl.pallas_call(
        _rating_pass_kernel,
        grid=(n_blocks,),
        in_specs=[
            pl.BlockSpec((1, RANK), lambda i: (0, 0)),
            pl.BlockSpec((N_ITEMS, RANK), lambda i: (0, 0)),
            pl.BlockSpec((BU, N_ITEMS), lambda i: (i, 0)),
            pl.BlockSpec((RANK, N_ITEMS), lambda i: (0, 0)),
        ],
        out_specs=pl.BlockSpec((BU, N_ITEMS), lambda i: (i, 0)),
        out_shape=jax.ShapeDtypeStruct((N_USERS, N_ITEMS), jnp.bfloat16),
        interpret=interpret,
    )(lam_row, item_sv, norm_adj, b_mat)

    return rating.astype(jnp.float16)
